# baseline (speedup 1.0000x reference)
import functools

import jax
import jax.numpy as jnp
from jax import lax
from jax.experimental import pallas as pl
from jax.experimental.pallas import tpu as pltpu
from jax.experimental.pallas import tpu_sc as plsc

D = 128
N = 100000
M = 25000

NCHUNK = 5
H = N // NCHUNK
NSC1 = 2

NS = 16
CH = 80
R = 4
G = 4
BPW = CH * R * G
HP = NS * BPW
G2 = G * NSC1
HP2 = NS * CH * R * G2

RB = 5000
EPS = 1e-5
_K0 = 0.7978845608028654
_K1 = 0.044715


def _sc_gather(table, idx3d, which_core, g):
    mesh = plsc.VectorSubcoreMesh(
        core_axis_name="c", subcore_axis_name="s", num_cores=2, num_subcores=NS
    )
    bpw = CH * R * g

    @functools.partial(
        pl.kernel,
        out_type=jax.ShapeDtypeStruct((NS * bpw, D), jnp.float32),
        mesh=mesh,
        scratch_types=[
            pltpu.VMEM((R * g, CH), jnp.int32),
            [pltpu.VMEM((CH, D), jnp.float32)] * (2 * R),
            [pltpu.SemaphoreType.DMA] * 4,
        ],
    )
    def k(table_hbm, idx_hbm, out_hbm, idx_v, bufs, sems):
        buf_a, buf_b = bufs[:R], bufs[R:]
        sem_ga, sem_gb, sem_oa, sem_ob = sems
        core = lax.axis_index("c")
        sub = lax.axis_index("s")

        @pl.when(core == which_core)
        def _work():
            pltpu.sync_copy(idx_hbm.at[sub], idx_v)
            base = sub * bpw

            def fire_g(buf, c, sem):
                return pltpu.async_copy(table_hbm.at[idx_v.at[c]], buf, sem)

            def fire_o(buf, c, sem):
                return pltpu.async_copy(
                    buf, out_hbm.at[pl.ds(base + c * CH, CH)], sem)

            def drain_g(buf, sem):
                pltpu.make_async_copy(table_hbm.at[pl.ds(0, CH)], buf, sem).wait()

            def drain_o(buf, sem):
                pltpu.make_async_copy(buf, out_hbm.at[pl.ds(base, CH)], sem).wait()

            for b in range(R):
                fire_g(buf_a[b], b, sem_ga)

            def body(u, carry):
                g0 = 2 * u
                g1 = g0 + 1
                for b in range(R):
                    drain_g(buf_a[b], sem_ga)
                outs_a = [fire_o(buf_a[b], g0 * R + b, sem_oa) for b in range(R)]
                @pl.when(u > 0)
                def _():
                    for b in range(R):
                        drain_o(buf_b[b], sem_ob)
                gb = [fire_g(buf_b[b], g1 * R + b, sem_gb) for b in range(R)]
                for d in gb:
                    d.wait()
                for b in range(R):
                    fire_o(buf_b[b], g1 * R + b, sem_ob)
                for d in outs_a:
                    d.wait()
                @pl.when(u + 1 < g // 2)
                def _():
                    for b in range(R):
                        fire_g(buf_a[b], (g0 + 2) * R + b, sem_ga)
                return carry

            lax.fori_loop(0, g // 2, body, 0)
            for b in range(R):
                drain_o(buf_b[b], sem_ob)

    return k(table, idx3d)


def _gelu(x):
    t = jnp.tanh(_K0 * (x + _K1 * x * x * x))
    return 0.5 * x * (1.0 + t)


def _tc_mlp(prev, xi, xi_row0, xs, chunk, w1a, w1b, b1, g1, be1, w2, b2, g2, be2):
    def body(*refs):
        if prev is not None:
            refs = refs[1:]
        (xi_ref, xs_ref, w1a_ref, w1b_ref, b1_ref, g1_ref, be1_ref,
         w2_ref, b2_ref, g2_ref, be2_ref, out_ref) = refs
        xb = xi_ref[...].astype(jnp.bfloat16)
        sb = xs_ref[...].astype(jnp.bfloat16)
        h = jnp.dot(xb, w1a_ref[...], preferred_element_type=jnp.float32)
        h = h + jnp.dot(sb, w1b_ref[...], preferred_element_type=jnp.float32)
        h = h + b1_ref[...]
        mu = jnp.mean(h, axis=-1, keepdims=True)
        c = h - mu
        var = jnp.mean(c * c, axis=-1, keepdims=True)
        h = c * lax.rsqrt(var + EPS) * g1_ref[...] + be1_ref[...]
        h = _gelu(h)
        h = jnp.dot(h.astype(jnp.bfloat16), w2_ref[...],
                    preferred_element_type=jnp.float32) + b2_ref[...]
        mu = jnp.mean(h, axis=-1, keepdims=True)
        c = h - mu
        var = jnp.mean(c * c, axis=-1, keepdims=True)
        h = c * lax.rsqrt(var + EPS) * g2_ref[...] + be2_ref[...]
        out_ref[...] = _gelu(h)

    blk0 = (chunk * H) // RB
    xblk0 = xi_row0 // RB
    xi_spec = pl.BlockSpec((RB, D), lambda i: (xblk0 + i, 0))
    xs_spec = pl.BlockSpec((RB, D), lambda i: (blk0 + i, 0))
    out_spec = pl.BlockSpec((RB, D), lambda i: (blk0 + i, 0))
    full = lambda shape: pl.BlockSpec(shape, lambda i: (0,) * len(shape))
    in_specs = [
        xi_spec,
        xs_spec,
        full((D, D)), full((D, D)), full((1, D)), full((1, D)), full((1, D)),
        full((D, D)), full((1, D)), full((1, D)), full((1, D)),
    ]
    args = [xi, xs, w1a, w1b, b1, g1, be1, w2, b2, g2, be2]
    io_aliases = {}
    if prev is not None:
        in_specs = [pl.BlockSpec(memory_space=pltpu.MemorySpace.HBM)] + in_specs
        args = [prev] + args
        io_aliases = {0: 0}
    return pl.pallas_call(
        body,
        grid=(H // RB,),
        in_specs=in_specs,
        out_specs=out_spec,
        out_shape=jax.ShapeDtypeStruct((N, D), jnp.float32),
        input_output_aliases=io_aliases,
    )(*args)


def kernel(x_down, x_skip, upsample_idx, W1, b1, g1, be1, W2, b2, g2, be2):
    idx = upsample_idx.astype(jnp.int32)
    w1a = W1[:D].astype(jnp.bfloat16)
    w1b = W1[D:].astype(jnp.bfloat16)
    w2 = W2.astype(jnp.bfloat16)
    b1r, g1r, be1r = b1.reshape(1, D), g1.reshape(1, D), be1.reshape(1, D)
    b2r, g2r, be2r = b2.reshape(1, D), g2.reshape(1, D), be2.reshape(1, D)

    nsc0 = NCHUNK - NSC1
    tail = idx[nsc0 * H :]
    tail = jnp.concatenate([tail, jnp.zeros((HP2 - tail.shape[0],), jnp.int32)])
    xi_tail = _sc_gather(x_down, tail.reshape(NS, R * G2, CH), 1, G2)
    pad = jnp.zeros((HP - H,), jnp.int32)
    xis = []
    for c in range(nsc0):
        part = jnp.concatenate([idx[c * H : (c + 1) * H], pad])
        xis.append(_sc_gather(x_down, part.reshape(NS, R * G, CH), 0, G))

    out = None
    for c in range(NCHUNK):
        if c < nsc0:
            xi, row0 = xis[c], 0
        else:
            xi, row0 = xi_tail, (c - nsc0) * H
        out = _tc_mlp(out, xi, row0, x_skip, c, w1a, w1b,
                      b1r, g1r, be1r, w2, b2r, g2r, be2r)
    return out


# speedup vs baseline: 1.0574x; 1.0574x over previous
import functools

import jax
import jax.numpy as jnp
from jax import lax
from jax.experimental import pallas as pl
from jax.experimental.pallas import tpu as pltpu
from jax.experimental.pallas import tpu_sc as plsc

D = 128
N = 100000
M = 25000

NCHUNK = 4
H = N // NCHUNK
NSC1 = 1

NS = 16
CH = 80
R = 5
G = 4
BPW = CH * R * G
HP = NS * BPW
G2 = G * NSC1
HP2 = NS * CH * R * G2

RB = 5000
EPS = 1e-5
_K0 = 0.7978845608028654
_K1 = 0.044715


def _sc_gather(table, idx3d, which_core, g):
    mesh = plsc.VectorSubcoreMesh(
        core_axis_name="c", subcore_axis_name="s", num_cores=2, num_subcores=NS
    )
    bpw = CH * R * g

    @functools.partial(
        pl.kernel,
        out_type=jax.ShapeDtypeStruct((NS * bpw, D), jnp.float32),
        mesh=mesh,
        scratch_types=[
            pltpu.VMEM((R * g, CH), jnp.int32),
            [pltpu.VMEM((CH, D), jnp.float32)] * (2 * R),
            [pltpu.SemaphoreType.DMA] * 4,
        ],
    )
    def k(table_hbm, idx_hbm, out_hbm, idx_v, bufs, sems):
        buf_a, buf_b = bufs[:R], bufs[R:]
        sem_ga, sem_gb, sem_oa, sem_ob = sems
        core = lax.axis_index("c")
        sub = lax.axis_index("s")

        @pl.when(core == which_core)
        def _work():
            pltpu.sync_copy(idx_hbm.at[sub], idx_v)
            base = sub * bpw

            def fire_g(buf, c, sem):
                return pltpu.async_copy(table_hbm.at[idx_v.at[c]], buf, sem)

            def fire_o(buf, c, sem):
                return pltpu.async_copy(
                    buf, out_hbm.at[pl.ds(base + c * CH, CH)], sem)

            def drain_g(buf, sem):
                pltpu.make_async_copy(table_hbm.at[pl.ds(0, CH)], buf, sem).wait()

            def drain_o(buf, sem):
                pltpu.make_async_copy(buf, out_hbm.at[pl.ds(base, CH)], sem).wait()

            for b in range(R):
                fire_g(buf_a[b], b, sem_ga)

            def body(u, carry):
                g0 = 2 * u
                g1 = g0 + 1
                for b in range(R):
                    drain_g(buf_a[b], sem_ga)
                outs_a = [fire_o(buf_a[b], g0 * R + b, sem_oa) for b in range(R)]
                @pl.when(u > 0)
                def _():
                    for b in range(R):
                        drain_o(buf_b[b], sem_ob)
                gb = [fire_g(buf_b[b], g1 * R + b, sem_gb) for b in range(R)]
                for d in gb:
                    d.wait()
                for b in range(R):
                    fire_o(buf_b[b], g1 * R + b, sem_ob)
                for d in outs_a:
                    d.wait()
                @pl.when(u + 1 < g // 2)
                def _():
                    for b in range(R):
                        fire_g(buf_a[b], (g0 + 2) * R + b, sem_ga)
                return carry

            lax.fori_loop(0, g // 2, body, 0)
            for b in range(R):
                drain_o(buf_b[b], sem_ob)

    return k(table, idx3d)


def _gelu(x):
    t = jnp.tanh(_K0 * (x + _K1 * x * x * x))
    return 0.5 * x * (1.0 + t)


def _tc_mlp(prev, xi, xi_row0, xs, chunk, w1a, w1b, b1, g1, be1, w2, b2, g2, be2):
    def body(*refs):
        if prev is not None:
            refs = refs[1:]
        (xi_ref, xs_ref, w1a_ref, w1b_ref, b1_ref, g1_ref, be1_ref,
         w2_ref, b2_ref, g2_ref, be2_ref, out_ref) = refs
        xb = xi_ref[...].astype(jnp.bfloat16)
        sb = xs_ref[...].astype(jnp.bfloat16)
        h = jnp.dot(xb, w1a_ref[...], preferred_element_type=jnp.float32)
        h = h + jnp.dot(sb, w1b_ref[...], preferred_element_type=jnp.float32)
        h = h + b1_ref[...]
        mu = jnp.mean(h, axis=-1, keepdims=True)
        c = h - mu
        var = jnp.mean(c * c, axis=-1, keepdims=True)
        h = c * lax.rsqrt(var + EPS) * g1_ref[...] + be1_ref[...]
        h = _gelu(h)
        h = jnp.dot(h.astype(jnp.bfloat16), w2_ref[...],
                    preferred_element_type=jnp.float32) + b2_ref[...]
        mu = jnp.mean(h, axis=-1, keepdims=True)
        c = h - mu
        var = jnp.mean(c * c, axis=-1, keepdims=True)
        h = c * lax.rsqrt(var + EPS) * g2_ref[...] + be2_ref[...]
        out_ref[...] = _gelu(h)

    blk0 = (chunk * H) // RB
    xblk0 = xi_row0 // RB
    xi_spec = pl.BlockSpec((RB, D), lambda i: (xblk0 + i, 0))
    xs_spec = pl.BlockSpec((RB, D), lambda i: (blk0 + i, 0))
    out_spec = pl.BlockSpec((RB, D), lambda i: (blk0 + i, 0))
    full = lambda shape: pl.BlockSpec(shape, lambda i: (0,) * len(shape))
    in_specs = [
        xi_spec,
        xs_spec,
        full((D, D)), full((D, D)), full((1, D)), full((1, D)), full((1, D)),
        full((D, D)), full((1, D)), full((1, D)), full((1, D)),
    ]
    args = [xi, xs, w1a, w1b, b1, g1, be1, w2, b2, g2, be2]
    io_aliases = {}
    if prev is not None:
        in_specs = [pl.BlockSpec(memory_space=pltpu.MemorySpace.HBM)] + in_specs
        args = [prev] + args
        io_aliases = {0: 0}
    return pl.pallas_call(
        body,
        grid=(H // RB,),
        in_specs=in_specs,
        out_specs=out_spec,
        out_shape=jax.ShapeDtypeStruct((N, D), jnp.float32),
        input_output_aliases=io_aliases,
    )(*args)


def kernel(x_down, x_skip, upsample_idx, W1, b1, g1, be1, W2, b2, g2, be2):
    idx = upsample_idx.astype(jnp.int32)
    w1a = W1[:D].astype(jnp.bfloat16)
    w1b = W1[D:].astype(jnp.bfloat16)
    w2 = W2.astype(jnp.bfloat16)
    b1r, g1r, be1r = b1.reshape(1, D), g1.reshape(1, D), be1.reshape(1, D)
    b2r, g2r, be2r = b2.reshape(1, D), g2.reshape(1, D), be2.reshape(1, D)

    nsc0 = NCHUNK - NSC1
    tail = idx[nsc0 * H :]
    tail = jnp.concatenate([tail, jnp.zeros((HP2 - tail.shape[0],), jnp.int32)])
    xi_tail = _sc_gather(x_down, tail.reshape(NS, R * G2, CH), 1, G2)
    pad = jnp.zeros((HP - H,), jnp.int32)
    xis = []
    for c in range(nsc0):
        part = jnp.concatenate([idx[c * H : (c + 1) * H], pad])
        xis.append(_sc_gather(x_down, part.reshape(NS, R * G, CH), 0, G))

    out = None
    for c in range(NCHUNK):
        if c < nsc0:
            xi, row0 = xis[c], 0
        else:
            xi, row0 = xi_tail, (c - nsc0) * H
        out = _tc_mlp(out, xi, row0, x_skip, c, w1a, w1b,
                      b1r, g1r, be1r, w2, b2r, g2r, be2r)
    return out


# speedup vs baseline: 1.0630x; 1.0053x over previous
import functools

import jax
import jax.numpy as jnp
from jax import lax
from jax.experimental import pallas as pl
from jax.experimental.pallas import tpu as pltpu
from jax.experimental.pallas import tpu_sc as plsc

D = 128
N = 100000
M = 25000

NS = 16
CH = 80
R = 4
CHUNKS = [
    (30000, 0, 6),
    (30000, 0, 6),
    (30000, 0, 6),
    (10000, 1, 2),
]

RB = 5000
EPS = 1e-5
_K0 = 0.7978845608028654
_K1 = 0.044715


def _sc_gather(table, idx3d, which_core, g):
    mesh = plsc.VectorSubcoreMesh(
        core_axis_name="c", subcore_axis_name="s", num_cores=2, num_subcores=NS
    )
    bpw = CH * R * g

    @functools.partial(
        pl.kernel,
        out_type=jax.ShapeDtypeStruct((NS * bpw, D), jnp.float32),
        mesh=mesh,
        scratch_types=[
            pltpu.VMEM((R * g, CH), jnp.int32),
            [pltpu.VMEM((CH, D), jnp.float32)] * (2 * R),
            [pltpu.SemaphoreType.DMA] * 4,
        ],
    )
    def k(table_hbm, idx_hbm, out_hbm, idx_v, bufs, sems):
        buf_a, buf_b = bufs[:R], bufs[R:]
        sem_ga, sem_gb, sem_oa, sem_ob = sems
        core = lax.axis_index("c")
        sub = lax.axis_index("s")

        @pl.when(core == which_core)
        def _work():
            pltpu.sync_copy(idx_hbm.at[sub], idx_v)
            base = sub * bpw

            def fire_g(buf, c, sem):
                return pltpu.async_copy(table_hbm.at[idx_v.at[c]], buf, sem)

            def fire_o(buf, c, sem):
                return pltpu.async_copy(
                    buf, out_hbm.at[pl.ds(base + c * CH, CH)], sem)

            def drain_g(buf, sem):
                pltpu.make_async_copy(table_hbm.at[pl.ds(0, CH)], buf, sem).wait()

            def drain_o(buf, sem):
                pltpu.make_async_copy(buf, out_hbm.at[pl.ds(base, CH)], sem).wait()

            for b in range(R):
                fire_g(buf_a[b], b, sem_ga)

            def body(u, carry):
                g0 = 2 * u
                g1 = g0 + 1
                for b in range(R):
                    drain_g(buf_a[b], sem_ga)
                outs_a = [fire_o(buf_a[b], g0 * R + b, sem_oa) for b in range(R)]
                @pl.when(u > 0)
                def _():
                    for b in range(R):
                        drain_o(buf_b[b], sem_ob)
                gb = [fire_g(buf_b[b], g1 * R + b, sem_gb) for b in range(R)]
                for d in gb:
                    d.wait()
                for b in range(R):
                    fire_o(buf_b[b], g1 * R + b, sem_ob)
                for d in outs_a:
                    d.wait()
                @pl.when(u + 1 < g // 2)
                def _():
                    for b in range(R):
                        fire_g(buf_a[b], (g0 + 2) * R + b, sem_ga)
                return carry

            lax.fori_loop(0, g // 2, body, 0)
            for b in range(R):
                drain_o(buf_b[b], sem_ob)

    return k(table, idx3d)


def _gelu(x):
    t = jnp.tanh(_K0 * (x + _K1 * x * x * x))
    return 0.5 * x * (1.0 + t)


def _tc_mlp(prev, xi, xs, row0, rows, w1a, w1b, b1, g1, be1, w2, b2, g2, be2):
    def body(*refs):
        if prev is not None:
            refs = refs[1:]
        (xi_ref, xs_ref, w1a_ref, w1b_ref, b1_ref, g1_ref, be1_ref,
         w2_ref, b2_ref, g2_ref, be2_ref, out_ref) = refs
        xb = xi_ref[...]
        sb = xs_ref[...].astype(jnp.bfloat16)
        h = jnp.dot(xb, w1a_ref[...], preferred_element_type=jnp.float32)
        h = h + jnp.dot(sb, w1b_ref[...], preferred_element_type=jnp.float32)
        h = h + b1_ref[...]
        mu = jnp.mean(h, axis=-1, keepdims=True)
        c = h - mu
        var = jnp.mean(c * c, axis=-1, keepdims=True)
        h = c * lax.rsqrt(var + EPS) * g1_ref[...] + be1_ref[...]
        h = _gelu(h)
        h = jnp.dot(h.astype(jnp.bfloat16), w2_ref[...],
                    preferred_element_type=jnp.float32) + b2_ref[...]
        mu = jnp.mean(h, axis=-1, keepdims=True)
        c = h - mu
        var = jnp.mean(c * c, axis=-1, keepdims=True)
        h = c * lax.rsqrt(var + EPS) * g2_ref[...] + be2_ref[...]
        out_ref[...] = _gelu(h)

    blk0 = row0 // RB
    xi_spec = pl.BlockSpec((RB, D), lambda i: (i, 0))
    xs_spec = pl.BlockSpec((RB, D), lambda i: (blk0 + i, 0))
    out_spec = pl.BlockSpec((RB, D), lambda i: (blk0 + i, 0))
    full = lambda shape: pl.BlockSpec(shape, lambda i: (0,) * len(shape))
    in_specs = [
        xi_spec,
        xs_spec,
        full((D, D)), full((D, D)), full((1, D)), full((1, D)), full((1, D)),
        full((D, D)), full((1, D)), full((1, D)), full((1, D)),
    ]
    args = [xi, xs, w1a, w1b, b1, g1, be1, w2, b2, g2, be2]
    io_aliases = {}
    if prev is not None:
        in_specs = [pl.BlockSpec(memory_space=pltpu.MemorySpace.HBM)] + in_specs
        args = [prev] + args
        io_aliases = {0: 0}
    return pl.pallas_call(
        body,
        grid=(rows // RB,),
        in_specs=in_specs,
        out_specs=out_spec,
        out_shape=jax.ShapeDtypeStruct((N, D), jnp.float32),
        input_output_aliases=io_aliases,
    )(*args)


def kernel(x_down, x_skip, upsample_idx, W1, b1, g1, be1, W2, b2, g2, be2):
    idx = upsample_idx.astype(jnp.int32)
    w1a = W1[:D].astype(jnp.bfloat16)
    w1b = W1[D:].astype(jnp.bfloat16)
    w2 = W2.astype(jnp.bfloat16)
    b1r, g1r, be1r = b1.reshape(1, D), g1.reshape(1, D), be1.reshape(1, D)
    b2r, g2r, be2r = b2.reshape(1, D), g2.reshape(1, D), be2.reshape(1, D)

    starts = []
    row0 = 0
    for rows, _, _ in CHUNKS:
        starts.append(row0)
        row0 += rows
    order = sorted(range(len(CHUNKS)), key=lambda c: -CHUNKS[c][1])
    xis = {}
    for c in order:
        rows, which, g = CHUNKS[c]
        hp = NS * CH * R * g
        part = idx[starts[c] : starts[c] + rows]
        part = jnp.concatenate([part, jnp.zeros((hp - rows,), jnp.int32)])
        xis[c] = _sc_gather(x_down, part.reshape(NS, R * g, CH), which, g)

    out = None
    for c in range(len(CHUNKS)):
        rows, _, _ = CHUNKS[c]
        out = _tc_mlp(out, xis[c], x_skip, starts[c], rows, w1a, w1b,
                      b1r, g1r, be1r, w2, b2r, g2r, be2r)
    return out


# speedup vs baseline: 1.7344x; 1.6316x over previous
import functools

import jax
import jax.numpy as jnp
from jax import lax
from jax.experimental import pallas as pl
from jax.experimental.pallas import tpu as pltpu
from jax.experimental.pallas import tpu_sc as plsc

D = 128
N = 100000
M = 25000

NS = 16
CH = 56
R = 7
CHUNKS = [
    (25000, 0, 4),
    (25000, 0, 4),
    (25000, 0, 4),
    (25000, 1, 4),
]

RB = 5000
EPS = 1e-5
_K0 = 0.7978845608028654
_K1 = 0.044715


def _sc_gather(table, idx3d, which_core, g):
    mesh = plsc.VectorSubcoreMesh(
        core_axis_name="c", subcore_axis_name="s", num_cores=2, num_subcores=NS
    )
    bpw = CH * R * g

    @functools.partial(
        pl.kernel,
        out_type=jax.ShapeDtypeStruct((NS * bpw, D), jnp.float32),
        mesh=mesh,
        scratch_types=[
            pltpu.VMEM((R * g, CH), jnp.int32),
            [pltpu.VMEM((CH, D), jnp.float32)] * (2 * R),
            [pltpu.SemaphoreType.DMA] * 4,
        ],
    )
    def k(table_hbm, idx_hbm, out_hbm, idx_v, bufs, sems):
        buf_a, buf_b = bufs[:R], bufs[R:]
        sem_ga, sem_gb, sem_oa, sem_ob = sems
        core = lax.axis_index("c")
        sub = lax.axis_index("s")

        @pl.when(core == which_core)
        def _work():
            pltpu.sync_copy(idx_hbm.at[sub], idx_v)
            base = sub * bpw

            def fire_g(buf, c, sem):
                return pltpu.async_copy(table_hbm.at[idx_v.at[c]], buf, sem)

            def fire_o(buf, c, sem):
                return pltpu.async_copy(
                    buf, out_hbm.at[pl.ds(base + c * CH, CH)], sem)

            def drain_g(buf, sem):
                pltpu.make_async_copy(table_hbm.at[pl.ds(0, CH)], buf, sem).wait()

            def drain_o(buf, sem):
                pltpu.make_async_copy(buf, out_hbm.at[pl.ds(base, CH)], sem).wait()

            for b in range(R):
                fire_g(buf_a[b], b, sem_ga)

            def body(u, carry):
                g0 = 2 * u
                g1 = g0 + 1
                for b in range(R):
                    drain_g(buf_a[b], sem_ga)
                outs_a = [fire_o(buf_a[b], g0 * R + b, sem_oa) for b in range(R)]
                @pl.when(u > 0)
                def _():
                    for b in range(R):
                        drain_o(buf_b[b], sem_ob)
                gb = [fire_g(buf_b[b], g1 * R + b, sem_gb) for b in range(R)]
                for d in gb:
                    d.wait()
                for b in range(R):
                    fire_o(buf_b[b], g1 * R + b, sem_ob)
                for d in outs_a:
                    d.wait()
                @pl.when(u + 1 < g // 2)
                def _():
                    for b in range(R):
                        fire_g(buf_a[b], (g0 + 2) * R + b, sem_ga)
                return carry

            lax.fori_loop(0, g // 2, body, 0)
            for b in range(R):
                drain_o(buf_b[b], sem_ob)

    return k(table, idx3d)


def _gelu(x):
    t = jnp.tanh(_K0 * (x + _K1 * x * x * x))
    return 0.5 * x * (1.0 + t)


def _tc_mlp(prev, xi, xs, row0, rows, w1a, w1b, b1, g1, be1, w2, b2, g2, be2):
    def body(*refs):
        if prev is not None:
            refs = refs[1:]
        (xi_ref, xs_ref, w1a_ref, w1b_ref, b1_ref, g1_ref, be1_ref,
         w2_ref, b2_ref, g2_ref, be2_ref, out_ref) = refs
        xb = xi_ref[...]
        sb = xs_ref[...].astype(jnp.bfloat16)
        h = jnp.dot(xb, w1a_ref[...], preferred_element_type=jnp.float32)
        h = h + jnp.dot(sb, w1b_ref[...], preferred_element_type=jnp.float32)
        h = h + b1_ref[...]
        mu = jnp.mean(h, axis=-1, keepdims=True)
        c = h - mu
        var = jnp.mean(c * c, axis=-1, keepdims=True)
        h = c * lax.rsqrt(var + EPS) * g1_ref[...] + be1_ref[...]
        h = _gelu(h)
        h = jnp.dot(h.astype(jnp.bfloat16), w2_ref[...],
                    preferred_element_type=jnp.float32) + b2_ref[...]
        mu = jnp.mean(h, axis=-1, keepdims=True)
        c = h - mu
        var = jnp.mean(c * c, axis=-1, keepdims=True)
        h = c * lax.rsqrt(var + EPS) * g2_ref[...] + be2_ref[...]
        out_ref[...] = _gelu(h)

    blk0 = row0 // RB
    xi_spec = pl.BlockSpec((RB, D), lambda i: (i, 0))
    xs_spec = pl.BlockSpec((RB, D), lambda i: (blk0 + i, 0))
    out_spec = pl.BlockSpec((RB, D), lambda i: (blk0 + i, 0))
    full = lambda shape: pl.BlockSpec(shape, lambda i: (0,) * len(shape))
    in_specs = [
        xi_spec,
        xs_spec,
        full((D, D)), full((D, D)), full((1, D)), full((1, D)), full((1, D)),
        full((D, D)), full((1, D)), full((1, D)), full((1, D)),
    ]
    args = [xi, xs, w1a, w1b, b1, g1, be1, w2, b2, g2, be2]
    io_aliases = {}
    if prev is not None:
        in_specs = [pl.BlockSpec(memory_space=pltpu.MemorySpace.HBM)] + in_specs
        args = [prev] + args
        io_aliases = {0: 0}
    return pl.pallas_call(
        body,
        grid=(rows // RB,),
        in_specs=in_specs,
        out_specs=out_spec,
        out_shape=jax.ShapeDtypeStruct((N, D), jnp.float32),
        input_output_aliases=io_aliases,
    )(*args)


def kernel(x_down, x_skip, upsample_idx, W1, b1, g1, be1, W2, b2, g2, be2):
    idx = upsample_idx.astype(jnp.int32)
    w1a = W1[:D].astype(jnp.bfloat16)
    w1b = W1[D:].astype(jnp.bfloat16)
    w2 = W2.astype(jnp.bfloat16)
    b1r, g1r, be1r = b1.reshape(1, D), g1.reshape(1, D), be1.reshape(1, D)
    b2r, g2r, be2r = b2.reshape(1, D), g2.reshape(1, D), be2.reshape(1, D)

    starts = []
    row0 = 0
    for rows, _, _ in CHUNKS:
        starts.append(row0)
        row0 += rows
    order = sorted(range(len(CHUNKS)), key=lambda c: -CHUNKS[c][1])
    xis = {}
    for c in order:
        rows, which, g = CHUNKS[c]
        hp = NS * CH * R * g
        part = idx[starts[c] : starts[c] + rows]
        part = jnp.concatenate([part, jnp.zeros((hp - rows,), jnp.int32)])
        xis[c] = _sc_gather(x_down, part.reshape(NS, R * g, CH), which, g)

    out = None
    for c in range(len(CHUNKS)):
        rows, _, _ = CHUNKS[c]
        out = _tc_mlp(out, xis[c], x_skip, starts[c], rows, w1a, w1b,
                      b1r, g1r, be1r, w2, b2r, g2r, be2r)
    return out
